# baseline (speedup 1.0000x reference)
import jax
import jax.numpy as jnp
from jax.experimental import pallas as pl
from jax.experimental.pallas import tpu as pltpu

_N_PRO = 8


def _gather_body(idx_ref, w_ref, o_ref, tbl_ref):
    bm, S = o_ref.shape[0], o_ref.shape[1]
    vc = w_ref.shape[0]
    i = pl.program_id(0)

    @pl.when(i < _N_PRO)
    def _relayout():
        rows = vc * 8
        tbl_ref[pl.ds(i * rows, rows)] = w_ref[...].reshape(rows, 1, -1)

    @pl.when(i >= _N_PRO)
    def _gather():
        for b in range(bm):
            for s in range(S):
                o_ref[b, s] = tbl_ref[idx_ref[b, s], 0]


def kernel(indices, weight):
    B, S = indices.shape
    V, D = weight.shape
    idx = indices.astype(jnp.int32)

    bm = 8
    n_steps = B // bm
    vb = (V // 8) // _N_PRO

    w3 = weight.reshape(V // 8, 8, D)

    table_bytes = V * D * jnp.dtype(weight.dtype).itemsize
    out_block_bytes = bm * S * D * jnp.dtype(weight.dtype).itemsize
    vmem_limit = int(min(table_bytes + 2 * (table_bytes // _N_PRO)
                         + 2 * out_block_bytes + (4 << 20),
                         128 * 1024 * 1024))

    return pl.pallas_call(
        _gather_body,
        out_shape=jax.ShapeDtypeStruct((B, S, D), weight.dtype),
        grid=(_N_PRO + n_steps,),
        in_specs=[
            pl.BlockSpec((bm, S),
                         lambda i: (jnp.maximum(i - _N_PRO, 0), 0),
                         memory_space=pltpu.SMEM),
            pl.BlockSpec((vb, 8, D),
                         lambda i: (jnp.minimum(i, _N_PRO - 1), 0, 0)),
        ],
        out_specs=pl.BlockSpec((bm, S, D),
                               lambda i: (jnp.maximum(i - _N_PRO, 0), 0, 0)),
        scratch_shapes=[pltpu.VMEM((V, 1, D), jnp.float32)],
        compiler_params=pltpu.CompilerParams(
            dimension_semantics=("arbitrary",),
            vmem_limit_bytes=vmem_limit,
        ),
    )(idx, w3)


# speedup vs baseline: 1.0101x; 1.0101x over previous
import jax
import jax.numpy as jnp
from jax.experimental import pallas as pl
from jax.experimental.pallas import tpu as pltpu


def _gather_body(idx_ref, w_ref, o_ref, tbl_ref):
    bm, S = o_ref.shape[0], o_ref.shape[1]
    V = tbl_ref.shape[0]

    @pl.when(pl.program_id(0) == 0)
    def _relayout():
        tbl_ref[...] = w_ref[...].reshape(V, 1, -1)

    for b in range(bm):
        for s in range(S):
            o_ref[b, s] = tbl_ref[idx_ref[b, s], 0]


def kernel(indices, weight):
    B, S = indices.shape
    V, D = weight.shape
    idx = indices.astype(jnp.int32)

    bm = 8
    n_steps = B // bm

    w3 = weight.reshape(V // 8, 8, D)

    table_bytes = V * D * jnp.dtype(weight.dtype).itemsize
    out_block_bytes = bm * S * D * jnp.dtype(weight.dtype).itemsize
    vmem_limit = int(min(2 * table_bytes + 2 * out_block_bytes + (4 << 20),
                         128 * 1024 * 1024))

    return pl.pallas_call(
        _gather_body,
        out_shape=jax.ShapeDtypeStruct((B, S, D), weight.dtype),
        grid=(n_steps,),
        in_specs=[
            pl.BlockSpec((bm, S), lambda i: (i, 0), memory_space=pltpu.SMEM),
            pl.BlockSpec((V // 8, 8, D), lambda i: (0, 0, 0)),
        ],
        out_specs=pl.BlockSpec((bm, S, D), lambda i: (i, 0, 0)),
        scratch_shapes=[pltpu.VMEM((V, 1, D), jnp.float32)],
        compiler_params=pltpu.CompilerParams(
            dimension_semantics=("arbitrary",),
            vmem_limit_bytes=vmem_limit,
        ),
    )(idx, w3)
